# baseline (speedup 1.0000x reference)
import functools

import jax
import jax.numpy as jnp
from jax import lax
from jax.experimental import pallas as pl
from jax.experimental.pallas import tpu as pltpu
from jax.experimental.pallas import tpu_sc as plsc

D_MODEL_ = 2048
N_EXP_ = 64
ROWS_ = 16384
BM_ = 2048
CH_ = 1
CROWS_ = ROWS_ // CH_

NC_, NS_, L_ = 2, 16, 16
NW_ = NC_ * NS_
RPT_ = CROWS_ // NW_
NG_ = RPT_ // L_


def _mm_body(x_ref, w_ref, b_ref, out_ref):
    lt = jax.lax.dot_general(
        w_ref[...], x_ref[...], (((1,), (1,)), ((), ())),
        preferred_element_type=jnp.float32,
    )
    out_ref[...] = lt + b_ref[...]


def _tc_logits_t(x_chunk, W, b2):
    return pl.pallas_call(
        _mm_body,
        grid=(CROWS_ // BM_,),
        in_specs=[
            pl.BlockSpec((BM_, D_MODEL_), lambda i: (i, 0)),
            pl.BlockSpec((N_EXP_, D_MODEL_), lambda i: (0, 0)),
            pl.BlockSpec((N_EXP_, 1), lambda i: (0, 0)),
        ],
        out_specs=pl.BlockSpec((N_EXP_, BM_), lambda i: (0, i)),
        out_shape=jax.ShapeDtypeStruct((N_EXP_, CROWS_), jnp.float32),
        compiler_params=pltpu.CompilerParams(
            dimension_semantics=("arbitrary",),
        ),
    )(x_chunk, W, b2)


_GCHUNK_ = 8


def _topk_body(lt_hbm, w_hbm, i_hbm, lt_v, w1_v, w2_v, i1_v, i2_v, sems):
    wid = lax.axis_index("s") * NC_ + lax.axis_index("c")
    base = wid * RPT_
    nchunk = NG_ // _GCHUNK_
    span = _GCHUNK_ * L_
    copies = [
        pltpu.async_copy(
            lt_hbm.at[:, pl.ds(base + c * span, span)],
            lt_v.at[:, pl.ds(c * span, span)],
            sems.at[c],
        )
        for c in range(nchunk)
    ]

    def group(g, carry):
        off = g * L_
        mask_hi = jnp.full((L_,), ~63, jnp.int32)
        c63 = jnp.full((L_,), 63, jnp.int32)

        def key(e):
            v = lt_v[e, pl.ds(off, L_)]
            vi = lax.bitcast_convert_type(v, jnp.int32)
            ki = (vi & mask_hi) | jnp.full((L_,), 63 - e, jnp.int32)
            return lax.bitcast_convert_type(ki, jnp.float32)

        k0 = key(0)
        k1 = key(1)
        gt = k1 > k0
        max1 = jnp.where(gt, k1, k0)
        max2 = jnp.where(gt, k0, k1)
        for e in range(2, N_EXP_):
            k = key(e)
            gt1 = k > max1
            max2 = jnp.where(gt1, max1, jnp.maximum(max2, k))
            max1 = jnp.maximum(max1, k)
        m1i = lax.bitcast_convert_type(max1, jnp.int32)
        m2i = lax.bitcast_convert_type(max2, jnp.int32)
        idx1 = c63 - (m1i & c63)
        idx2 = c63 - (m2i & c63)
        e2 = jnp.exp(max2 - max1)
        w1 = 1.0 / (1.0 + e2)
        w2 = 1.0 - w1
        w1_v[pl.ds(off, L_)] = w1
        w2_v[pl.ds(off, L_)] = w2
        i1_v[pl.ds(off, L_)] = idx1
        i2_v[pl.ds(off, L_)] = idx2
        return carry

    for c in range(nchunk):
        copies[c].wait()
        lax.fori_loop(c * _GCHUNK_, (c + 1) * _GCHUNK_, group, 0)

    pltpu.sync_copy(w1_v, w_hbm.at[0, pl.ds(base, RPT_)])
    pltpu.sync_copy(w2_v, w_hbm.at[1, pl.ds(base, RPT_)])
    pltpu.sync_copy(i1_v, i_hbm.at[0, pl.ds(base, RPT_)])
    pltpu.sync_copy(i2_v, i_hbm.at[1, pl.ds(base, RPT_)])


_topk_sc = functools.partial(
    pl.kernel,
    out_type=[
        jax.ShapeDtypeStruct((2, CROWS_), jnp.float32),
        jax.ShapeDtypeStruct((2, CROWS_), jnp.int32),
    ],
    mesh=plsc.VectorSubcoreMesh(core_axis_name="c", subcore_axis_name="s"),
    scratch_types=[
        pltpu.VMEM((N_EXP_, RPT_), jnp.float32),
        pltpu.VMEM((RPT_,), jnp.float32),
        pltpu.VMEM((RPT_,), jnp.float32),
        pltpu.VMEM((RPT_,), jnp.int32),
        pltpu.VMEM((RPT_,), jnp.int32),
        pltpu.SemaphoreType.DMA((NG_ // _GCHUNK_,)),
    ],
)(_topk_body)


def kernel(x, W, b):
    x_flat = x.reshape(ROWS_, D_MODEL_)
    b2 = b.reshape(N_EXP_, 1)
    w_parts, i_parts = [], []
    for c in range(CH_):
        x_chunk = lax.slice_in_dim(x_flat, c * CROWS_, (c + 1) * CROWS_, axis=0)
        lt = _tc_logits_t(x_chunk, W, b2)
        w_pair, i_pair = _topk_sc(lt)
        w_parts.append(w_pair)
        i_parts.append(i_pair)
    w = jnp.concatenate(w_parts, axis=1).T
    i = jnp.concatenate(i_parts, axis=1).T
    return (w, i)


# speedup vs baseline: 1.0231x; 1.0231x over previous
import functools

import jax
import jax.numpy as jnp
from jax import lax
from jax.experimental import pallas as pl
from jax.experimental.pallas import tpu as pltpu
from jax.experimental.pallas import tpu_sc as plsc

D_MODEL_ = 2048
N_EXP_ = 64
ROWS_ = 16384
BM_ = 2048
CH_ = 1
CROWS_ = ROWS_ // CH_

NC_, NS_, L_ = 2, 16, 16
NW_ = NC_ * NS_
RPT_ = CROWS_ // NW_
NG_ = RPT_ // L_


def _mm_body(x_ref, w_ref, b_ref, out_ref):
    lt = jax.lax.dot_general(
        w_ref[...], x_ref[...], (((1,), (1,)), ((), ())),
        preferred_element_type=jnp.float32,
    )
    out_ref[...] = lt + b_ref[...]


def _tc_logits_t(x_chunk, W, b2):
    return pl.pallas_call(
        _mm_body,
        grid=(CROWS_ // BM_,),
        in_specs=[
            pl.BlockSpec((BM_, D_MODEL_), lambda i: (i, 0)),
            pl.BlockSpec((N_EXP_, D_MODEL_), lambda i: (0, 0)),
            pl.BlockSpec((N_EXP_, 1), lambda i: (0, 0)),
        ],
        out_specs=pl.BlockSpec((N_EXP_, BM_), lambda i: (0, i)),
        out_shape=jax.ShapeDtypeStruct((N_EXP_, CROWS_), jnp.float32),
        compiler_params=pltpu.CompilerParams(
            dimension_semantics=("arbitrary",),
        ),
    )(x_chunk, W, b2)


_GCHUNK_ = 8


def _topk_body(lt_hbm, w_hbm, i_hbm, lt_v, w1_v, w2_v, i1_v, i2_v):
    wid = lax.axis_index("s") * NC_ + lax.axis_index("c")
    base = wid * RPT_
    nchunk = NG_ // _GCHUNK_
    span = _GCHUNK_ * L_
    pltpu.sync_copy(lt_hbm.at[:, pl.ds(base, RPT_)], lt_v)

    def group(g, carry):
        off = g * L_
        v0 = lt_v[0, pl.ds(off, L_)]
        v1 = lt_v[1, pl.ds(off, L_)]
        c0 = jnp.zeros((L_,), jnp.int32)
        c1 = jnp.ones((L_,), jnp.int32)
        gt = v1 > v0
        max1 = jnp.where(gt, v1, v0)
        idx1 = jnp.where(gt, c1, c0)
        max2 = jnp.where(gt, v0, v1)
        idx2 = jnp.where(gt, c0, c1)
        for e in range(2, N_EXP_):
            v = lt_v[e, pl.ds(off, L_)]
            ev = jnp.full((L_,), e, jnp.int32)
            gt1 = v > max1
            gt2 = v > max2
            idx2 = jnp.where(gt1, idx1, jnp.where(gt2, ev, idx2))
            max2 = jnp.where(gt1, max1, jnp.maximum(max2, v))
            idx1 = jnp.where(gt1, ev, idx1)
            max1 = jnp.maximum(max1, v)
        e2 = jnp.exp(max2 - max1)
        w1 = 1.0 / (1.0 + e2)
        w2 = 1.0 - w1
        w1_v[pl.ds(off, L_)] = w1
        w2_v[pl.ds(off, L_)] = w2
        i1_v[pl.ds(off, L_)] = idx1
        i2_v[pl.ds(off, L_)] = idx2
        return carry

    lax.fori_loop(0, NG_, group, 0)

    pltpu.sync_copy(w1_v, w_hbm.at[0, pl.ds(base, RPT_)])
    pltpu.sync_copy(w2_v, w_hbm.at[1, pl.ds(base, RPT_)])
    pltpu.sync_copy(i1_v, i_hbm.at[0, pl.ds(base, RPT_)])
    pltpu.sync_copy(i2_v, i_hbm.at[1, pl.ds(base, RPT_)])


_topk_sc = functools.partial(
    pl.kernel,
    out_type=[
        jax.ShapeDtypeStruct((2, CROWS_), jnp.float32),
        jax.ShapeDtypeStruct((2, CROWS_), jnp.int32),
    ],
    mesh=plsc.VectorSubcoreMesh(core_axis_name="c", subcore_axis_name="s"),
    scratch_types=[
        pltpu.VMEM((N_EXP_, RPT_), jnp.float32),
        pltpu.VMEM((RPT_,), jnp.float32),
        pltpu.VMEM((RPT_,), jnp.float32),
        pltpu.VMEM((RPT_,), jnp.int32),
        pltpu.VMEM((RPT_,), jnp.int32),
    ],
)(_topk_body)


def kernel(x, W, b):
    x_flat = x.reshape(ROWS_, D_MODEL_)
    b2 = b.reshape(N_EXP_, 1)
    w_parts, i_parts = [], []
    for c in range(CH_):
        x_chunk = lax.slice_in_dim(x_flat, c * CROWS_, (c + 1) * CROWS_, axis=0)
        lt = _tc_logits_t(x_chunk, W, b2)
        w_pair, i_pair = _topk_sc(lt)
        w_parts.append(w_pair)
        i_parts.append(i_pair)
    w = jnp.concatenate(w_parts, axis=1).T
    i = jnp.concatenate(i_parts, axis=1).T
    return (w, i)
